# Initial kernel scaffold; baseline (speedup 1.0000x reference)
import jax
import jax.numpy as jnp
from jax.experimental import pallas as pl


def kernel(x):
    raise NotImplementedError("write your pallas kernel here")


# speedup vs baseline: 3.7782x; 3.7782x over previous
import jax
import jax.numpy as jnp
from jax import lax
from jax.experimental import pallas as pl
from jax.experimental.pallas import tpu as pltpu

N = 8192
DIM = 512
K = 16
BR = 256
BC = 1024
PAD = 128
W = PAD + BC
NR = N // BR
NC = N // BC
INF = float("inf")
BIG = 2**30


def _norms_body(x_ref, out_ref):
    x = x_ref[...]
    out_ref[...] = jnp.sum(x * x, axis=1, keepdims=True)


def _knn_body(xi_ref, xjt_ref, x2r_ref, x2c_ref, out_ref,
              mv_ref, mi_ref, bv_ref, bi_ref):
    j = pl.program_id(1)

    @pl.when(j == 0)
    def _init():
        mv_ref[:, :PAD] = jnp.full((BR, PAD), INF, jnp.float32)
        mi_ref[:, :PAD] = jnp.full((BR, PAD), BIG, jnp.int32)
        bv_ref[...] = jnp.full((BR, PAD), INF, jnp.float32)
        bi_ref[...] = jnp.full((BR, PAD), BIG, jnp.int32)

    mm = jnp.dot(xi_ref[...], xjt_ref[...],
                 preferred_element_type=jnp.float32)
    d = (x2r_ref[...] + x2c_ref[...]) - 2.0 * mm
    mv_ref[:, PAD:] = d
    mi_ref[:, PAD:] = lax.broadcasted_iota(jnp.int32, (BR, BC), 1) + j * BC

    lanes = lax.broadcasted_iota(jnp.int32, (BR, PAD), 1)

    def pass_t(t, carry):
        mv = mv_ref[...]
        mi = mi_ref[...]
        m = jnp.min(mv, axis=1, keepdims=True)
        am = jnp.min(jnp.where(mv == m, mi, BIG), axis=1, keepdims=True)
        mv_ref[...] = jnp.where(mi == am, INF, mv)
        bv_ref[...] = jnp.where(lanes == t, m, bv_ref[...])
        bi_ref[...] = jnp.where(lanes == t, am, bi_ref[...])
        return carry

    lax.fori_loop(0, K, pass_t, 0)

    mv_ref[:, :PAD] = bv_ref[...]
    mi_ref[:, :PAD] = bi_ref[...]

    @pl.when(j == NC - 1)
    def _emit():
        out_ref[...] = bi_ref[:, :K]


def kernel(x):
    x2r = pl.pallas_call(
        _norms_body,
        out_shape=jax.ShapeDtypeStruct((N, 1), jnp.float32),
    )(x)
    xt = x.T
    x2c = x2r.T
    idx = pl.pallas_call(
        _knn_body,
        grid=(NR, NC),
        in_specs=[
            pl.BlockSpec((BR, DIM), lambda i, j: (i, 0)),
            pl.BlockSpec((DIM, BC), lambda i, j: (0, j)),
            pl.BlockSpec((BR, 1), lambda i, j: (i, 0)),
            pl.BlockSpec((1, BC), lambda i, j: (0, j)),
        ],
        out_specs=pl.BlockSpec((BR, K), lambda i, j: (i, 0)),
        out_shape=jax.ShapeDtypeStruct((N, K), jnp.int32),
        scratch_shapes=[
            pltpu.VMEM((BR, W), jnp.float32),
            pltpu.VMEM((BR, W), jnp.int32),
            pltpu.VMEM((BR, PAD), jnp.float32),
            pltpu.VMEM((BR, PAD), jnp.int32),
        ],
    )(x, xt, x2r, x2c)
    src = idx.reshape(-1).astype(jnp.int64)
    dst = jnp.repeat(jnp.arange(N, dtype=jnp.int64), K)
    return src, dst


# speedup vs baseline: 4.4371x; 1.1744x over previous
import jax
import jax.numpy as jnp
from jax import lax
from jax.experimental import pallas as pl
from jax.experimental.pallas import tpu as pltpu

N = 8192
DIM = 512
K = 16
BR = 256
BC = 2048
PAD = 128
NR = N // BR
NC = N // BC
INF = float("inf")
BIG = 2**30


def _norms_body(x_ref, out_ref):
    x = x_ref[...]
    out_ref[...] = jnp.sum(x * x, axis=1, keepdims=True)


def _knn_body(xi_ref, xjt_ref, x2r_ref, x2c_ref, out_ref,
              d_ref, bv_ref, bi_ref, nv_ref, ni_ref):
    j = pl.program_id(1)

    @pl.when(j == 0)
    def _init():
        bv_ref[...] = jnp.full((BR, PAD), INF, jnp.float32)
        bi_ref[...] = jnp.full((BR, PAD), BIG, jnp.int32)
        nv_ref[...] = jnp.full((BR, PAD), INF, jnp.float32)
        ni_ref[...] = jnp.full((BR, PAD), BIG, jnp.int32)

    mm = jnp.dot(xi_ref[...], xjt_ref[...],
                 preferred_element_type=jnp.float32)
    d_ref[...] = (x2r_ref[...] + x2c_ref[...]) - 2.0 * mm

    joff = j * BC
    lanes = lax.broadcasted_iota(jnp.int32, (BR, PAD), 1)

    def pass_t(t, carry):
        dv = d_ref[...]
        bvv = bv_ref[...]
        biv = bi_ref[...]
        iota = lax.broadcasted_iota(jnp.int32, (BR, BC), 1)
        m = jnp.minimum(jnp.min(dv, axis=1, keepdims=True),
                        jnp.min(bvv, axis=1, keepdims=True))
        amd = jnp.min(jnp.where(dv == m, iota, BIG),
                      axis=1, keepdims=True) + joff
        amp = jnp.min(jnp.where(bvv == m, biv, BIG),
                      axis=1, keepdims=True)
        am = jnp.minimum(amp, amd)
        d_ref[...] = jnp.where(iota == (am - joff), INF, dv)
        bv_ref[...] = jnp.where(biv == am, INF, bvv)
        nv_ref[...] = jnp.where(lanes == t, m, nv_ref[...])
        ni_ref[...] = jnp.where(lanes == t, am, ni_ref[...])
        return carry

    lax.fori_loop(0, K, pass_t, 0)

    bv_ref[...] = nv_ref[...]
    bi_ref[...] = ni_ref[...]
    nv_ref[...] = jnp.full((BR, PAD), INF, jnp.float32)
    ni_ref[...] = jnp.full((BR, PAD), BIG, jnp.int32)

    @pl.when(j == NC - 1)
    def _emit():
        out_ref[...] = bi_ref[:, :K]


def kernel(x):
    x2r = pl.pallas_call(
        _norms_body,
        out_shape=jax.ShapeDtypeStruct((N, 1), jnp.float32),
    )(x)
    xt = x.T
    x2c = x2r.T
    idx = pl.pallas_call(
        _knn_body,
        grid=(NR, NC),
        in_specs=[
            pl.BlockSpec((BR, DIM), lambda i, j: (i, 0)),
            pl.BlockSpec((DIM, BC), lambda i, j: (0, j)),
            pl.BlockSpec((BR, 1), lambda i, j: (i, 0)),
            pl.BlockSpec((1, BC), lambda i, j: (0, j)),
        ],
        out_specs=pl.BlockSpec((BR, K), lambda i, j: (i, 0)),
        out_shape=jax.ShapeDtypeStruct((N, K), jnp.int32),
        scratch_shapes=[
            pltpu.VMEM((BR, BC), jnp.float32),
            pltpu.VMEM((BR, PAD), jnp.float32),
            pltpu.VMEM((BR, PAD), jnp.int32),
            pltpu.VMEM((BR, PAD), jnp.float32),
            pltpu.VMEM((BR, PAD), jnp.int32),
        ],
    )(x, xt, x2r, x2c)
    src = idx.reshape(-1).astype(jnp.int64)
    dst = jnp.repeat(jnp.arange(N, dtype=jnp.int64), K)
    return src, dst


# speedup vs baseline: 5.3240x; 1.1999x over previous
import jax
import jax.numpy as jnp
from jax import lax
from jax.experimental import pallas as pl
from jax.experimental.pallas import tpu as pltpu

N = 8192
DIM = 512
K = 16
BR = 256
BC = 2048
TL = 128
NT = BC // TL
PAD = 128
UW = 4 * TL
NR = N // BR
NC = N // BC
INF = float("inf")
BIG = 2**30


def _norms_body(x_ref, out_ref):
    x = x_ref[...]
    out_ref[...] = jnp.sum(x * x, axis=1, keepdims=True)


def _merge_sorted3(a, b):
    (av, ai, acnt), (bv, bi, bcnt) = a, b
    c0 = av[0] <= bv[0]
    o0v = jnp.where(c0, av[0], bv[0])
    o0i = jnp.where(c0, ai[0], bi[0])
    hv = jnp.where(c0, bv[0], av[0])
    hi = jnp.where(c0, bi[0], ai[0])
    c1 = av[1] <= bv[1]
    m1v = jnp.where(c1, av[1], bv[1])
    m1i = jnp.where(c1, ai[1], bi[1])
    c2 = hv <= m1v
    o1v = jnp.where(c2, hv, m1v)
    o1i = jnp.where(c2, hi, m1i)
    h2v = jnp.where(c2, m1v, hv)
    h2i = jnp.where(c2, m1i, hi)
    c3 = av[2] <= bv[2]
    m2v = jnp.where(c3, av[2], bv[2])
    m2i = jnp.where(c3, ai[2], bi[2])
    c4 = h2v <= m2v
    o2v = jnp.where(c4, h2v, m2v)
    o2i = jnp.where(c4, h2i, m2i)
    return ([o0v, o1v, o2v], [o0i, o1i, o2i], acnt + bcnt)


def _knn_body(xi_ref, xjt_ref, x2r_ref, x2c_ref, out_ref,
              d_ref, u_ref, ui_ref, bv_ref, bi_ref, nv_ref, ni_ref):
    j = pl.program_id(1)

    @pl.when(j == 0)
    def _init():
        bv_ref[...] = jnp.full((BR, PAD), INF, jnp.float32)
        bi_ref[...] = jnp.full((BR, PAD), BIG, jnp.int32)
        nv_ref[...] = jnp.full((BR, PAD), INF, jnp.float32)
        ni_ref[...] = jnp.full((BR, PAD), BIG, jnp.int32)

    mm = jnp.dot(xi_ref[...], xjt_ref[...],
                 preferred_element_type=jnp.float32)
    d_ref[...] = (x2r_ref[...] + x2c_ref[...]) - 2.0 * mm

    joff = j * BC
    lanes = lax.broadcasted_iota(jnp.int32, (BR, PAD), 1)

    tau = bv_ref[:, K - 1:K]
    dv = d_ref[...]
    leaves = []
    for k in range(NT):
        v = dv[:, k * TL:(k + 1) * TL]
        keep = v < tau
        fv = jnp.where(keep, v, INF)
        fi = lanes + (joff + k * TL)
        cnt = keep.astype(jnp.int32)
        big = jnp.full((BR, TL), INF, jnp.float32)
        bigi = jnp.full((BR, TL), BIG, jnp.int32)
        leaves.append(([fv, big, big], [fi, bigi, bigi], cnt))
    while len(leaves) > 1:
        half = len(leaves) // 2
        leaves = [_merge_sorted3(leaves[s], leaves[s + half])
                  for s in range(half)]
    (s_v, s_i, cnt) = leaves[0]
    overflow = jnp.max(cnt) > 3

    @pl.when(jnp.logical_not(overflow))
    def _fast():
        u_ref[:, 0 * TL:1 * TL] = s_v[0]
        u_ref[:, 1 * TL:2 * TL] = s_v[1]
        u_ref[:, 2 * TL:3 * TL] = s_v[2]
        u_ref[:, 3 * TL:4 * TL] = bv_ref[...]
        ui_ref[:, 0 * TL:1 * TL] = s_i[0]
        ui_ref[:, 1 * TL:2 * TL] = s_i[1]
        ui_ref[:, 2 * TL:3 * TL] = s_i[2]
        ui_ref[:, 3 * TL:4 * TL] = bi_ref[...]

        def pass_u(t, carry):
            uv = u_ref[...]
            uiv = ui_ref[...]
            m = jnp.min(uv, axis=1, keepdims=True)
            am = jnp.min(jnp.where(uv == m, uiv, BIG),
                         axis=1, keepdims=True)
            u_ref[...] = jnp.where(uiv == am, INF, uv)
            nv_ref[...] = jnp.where(lanes == t, m, nv_ref[...])
            ni_ref[...] = jnp.where(lanes == t, am, ni_ref[...])
            return carry

        lax.fori_loop(0, K, pass_u, 0)

    @pl.when(overflow)
    def _slow():
        def pass_t(t, carry):
            dvv = d_ref[...]
            bvv = bv_ref[...]
            biv = bi_ref[...]
            iota = lax.broadcasted_iota(jnp.int32, (BR, BC), 1)
            m = jnp.minimum(jnp.min(dvv, axis=1, keepdims=True),
                            jnp.min(bvv, axis=1, keepdims=True))
            amd = jnp.min(jnp.where(dvv == m, iota, BIG),
                          axis=1, keepdims=True) + joff
            amp = jnp.min(jnp.where(bvv == m, biv, BIG),
                          axis=1, keepdims=True)
            am = jnp.minimum(amp, amd)
            d_ref[...] = jnp.where(iota == (am - joff), INF, dvv)
            bv_ref[...] = jnp.where(biv == am, INF, bvv)
            nv_ref[...] = jnp.where(lanes == t, m, nv_ref[...])
            ni_ref[...] = jnp.where(lanes == t, am, ni_ref[...])
            return carry

        lax.fori_loop(0, K, pass_t, 0)

    bv_ref[...] = nv_ref[...]
    bi_ref[...] = ni_ref[...]
    nv_ref[...] = jnp.full((BR, PAD), INF, jnp.float32)
    ni_ref[...] = jnp.full((BR, PAD), BIG, jnp.int32)

    @pl.when(j == NC - 1)
    def _emit():
        out_ref[...] = bi_ref[:, :K]


def kernel(x):
    x2r = pl.pallas_call(
        _norms_body,
        out_shape=jax.ShapeDtypeStruct((N, 1), jnp.float32),
    )(x)
    xt = x.T
    x2c = x2r.T
    idx = pl.pallas_call(
        _knn_body,
        grid=(NR, NC),
        in_specs=[
            pl.BlockSpec((BR, DIM), lambda i, j: (i, 0)),
            pl.BlockSpec((DIM, BC), lambda i, j: (0, j)),
            pl.BlockSpec((BR, 1), lambda i, j: (i, 0)),
            pl.BlockSpec((1, BC), lambda i, j: (0, j)),
        ],
        out_specs=pl.BlockSpec((BR, K), lambda i, j: (i, 0)),
        out_shape=jax.ShapeDtypeStruct((N, K), jnp.int32),
        scratch_shapes=[
            pltpu.VMEM((BR, BC), jnp.float32),
            pltpu.VMEM((BR, UW), jnp.float32),
            pltpu.VMEM((BR, UW), jnp.int32),
            pltpu.VMEM((BR, PAD), jnp.float32),
            pltpu.VMEM((BR, PAD), jnp.int32),
            pltpu.VMEM((BR, PAD), jnp.float32),
            pltpu.VMEM((BR, PAD), jnp.int32),
        ],
    )(x, xt, x2r, x2c)
    src = idx.reshape(-1).astype(jnp.int64)
    dst = jnp.repeat(jnp.arange(N, dtype=jnp.int64), K)
    return src, dst
